# baseline (speedup 1.0000x reference)
import functools

import jax
import jax.numpy as jnp
from jax import lax
from jax.experimental import pallas as pl
from jax.experimental.pallas import tpu as pltpu
from jax.experimental.pallas import tpu_sc as plsc

NUM_ITEM = 1000000
HIDDEN = 64
SEQ = 200
BATCH = 4096
PAD_IDX = 3

NC = 2
NS = 16
LANES = 16
NW = NC * NS
N = BATCH * SEQ
ROWS_PW = N // NW
GSZ = 128
BT = BATCH // GSZ
NG = ROWS_PW // GSZ
PADH = 2 * HIDDEN


def _emb_body(xf, table, posf, out3, mask_out, idx_all, pos_v,
              gbufs, obufs, mask_bs, gsems, osems):
    wid = lax.axis_index("s") * NC + lax.axis_index("c")
    base = wid * ROWS_PW
    g0 = wid * NG
    pltpu.sync_copy(xf.at[pl.ds(base, ROWS_PW)], idx_all)
    pltpu.sync_copy(posf, pos_v)

    def start_gather(l, gbuf, sem):
        pltpu.async_copy(table.at[idx_all.at[pl.ds(l * GSZ, GSZ)]],
                         gbuf, sem)

    def drain_gather(l, gbuf, sem):
        pltpu.make_async_copy(table.at[idx_all.at[pl.ds(l * GSZ, GSZ)]],
                              gbuf, sem).wait()

    def out_copy(l, obuf, mbuf, sem):
        g = g0 + l
        s = g // BT
        bt = g - s * BT
        return (
            pltpu.make_async_copy(
                obuf, out3.at[s, :, pl.ds(bt * GSZ, GSZ)], sem),
            pltpu.make_async_copy(
                mbuf, mask_out.at[pl.ds(base + l * GSZ, GSZ)], sem),
        )

    iota16 = lax.iota(jnp.int32, LANES)
    start_gather(0, gbufs[0], gsems[0])

    def pair_body(cc, carry):
        for b in range(2):
            l = cc * 2 + b
            gbuf, obuf, mbuf = gbufs[b], obufs[b], mask_bs[b]
            nb = 1 - b
            @pl.when(l >= 1)
            def _():
                for h in out_copy(l - 1, obufs[nb], mask_bs[nb], osems[nb]):
                    h.wait()
            @pl.when(l + 1 < NG)
            def _():
                start_gather(l + 1, gbufs[nb], gsems[nb])
            npvs = []
            for i in range(GSZ // LANES):
                v = idx_all[pl.ds(l * GSZ + i * LANES, LANES)]
                ispad = v == PAD_IDX
                mbuf[pl.ds(i * LANES, LANES)] = jnp.where(ispad, 1, 0)
                npvs.append(jnp.where(ispad, 0.0, 1.0))
            drain_gather(l, gbuf, gsems[b])
            s = (g0 + l) // BT
            pbase = s * HIDDEN

            def h_body(h, carry2):
                psc = pos_v[pl.ds(pbase + h, LANES)][0]
                for i in range(GSZ // LANES):
                    colv = plsc.load_gather(
                        gbuf, [iota16 + i * LANES,
                               jnp.broadcast_to(h, (LANES,))])
                    obuf[h, pl.ds(i * LANES, LANES)] = (
                        (colv + psc) * npvs[i])
                return carry2

            lax.fori_loop(0, HIDDEN, h_body, 0, unroll=False)
            for h in out_copy(l, obuf, mbuf, osems[b]):
                h.start()
        return carry

    lax.fori_loop(0, NG // 2, pair_body, 0, unroll=False)
    for h in out_copy(NG - 1, obufs[1], mask_bs[1], osems[1]):
        h.wait()


_emb_call = pl.kernel(
    _emb_body,
    out_type=[
        jax.ShapeDtypeStruct((SEQ, HIDDEN, BATCH), jnp.float32),
        jax.ShapeDtypeStruct((N,), jnp.int32),
    ],
    mesh=plsc.VectorSubcoreMesh(
        core_axis_name="c", subcore_axis_name="s", num_cores=NC,
        num_subcores=NS),
    scratch_types=[
        pltpu.VMEM((ROWS_PW,), jnp.int32),
        pltpu.VMEM((SEQ * HIDDEN + LANES,), jnp.float32),
        [pltpu.VMEM((GSZ, PADH), jnp.float32)] * 2,
        [pltpu.VMEM((HIDDEN, GSZ), jnp.float32)] * 2,
        [pltpu.VMEM((GSZ,), jnp.int32)] * 2,
        [pltpu.SemaphoreType.DMA] * 2,
        [pltpu.SemaphoreType.DMA] * 2,
    ],
    compiler_params=pltpu.CompilerParams(needs_layout_passes=False),
)


def kernel(x, item_table, pos_table):
    xf = x.T.reshape(N)
    tp = jnp.pad(item_table, ((0, 0), (0, PADH - HIDDEN)))
    posf = jnp.pad(pos_table.reshape(SEQ * HIDDEN), (0, LANES))
    out3, mask_i32 = _emb_call(xf, tp, posf)
    input_emb = out3.transpose(2, 0, 1)
    pad_masking = mask_i32.reshape(SEQ, BATCH).T.astype(bool)
    return (input_emb, pad_masking)


# speedup vs baseline: 1.1666x; 1.1666x over previous
import functools

import jax
import jax.numpy as jnp
from jax import lax
from jax.experimental import pallas as pl
from jax.experimental.pallas import tpu as pltpu
from jax.experimental.pallas import tpu_sc as plsc

NUM_ITEM = 1000000
HIDDEN = 64
SEQ = 200
BATCH = 4096
PAD_IDX = 3

NC = 2
NS = 16
LANES = 16
NW = NC * NS
N = BATCH * SEQ
ROWS_PW = N // NW
GSZ = 128
BT = BATCH // GSZ
NG = ROWS_PW // GSZ
PADH = 2 * HIDDEN
OST = GSZ + 1
NBUF = 2
RUNROLL = 4


def _emb_body(xf, table, posf, out3, mask_out, idx_all, pos_v,
              gbufs, obufs, mask_bs, np_b, gsems, osems):
    wid = lax.axis_index("s") * NC + lax.axis_index("c")
    base = wid * ROWS_PW
    g0 = wid * NG
    pltpu.sync_copy(xf.at[pl.ds(base, ROWS_PW)], idx_all)
    pltpu.sync_copy(posf, pos_v)

    def start_gather(l, gbuf, sem):
        pltpu.async_copy(table.at[idx_all.at[pl.ds(l * GSZ, GSZ)]],
                         gbuf, sem)

    def drain_gather(l, gbuf, sem):
        pltpu.make_async_copy(table.at[idx_all.at[pl.ds(l * GSZ, GSZ)]],
                              gbuf, sem).wait()

    def out_copy(l, obuf, mbuf, sem):
        g = g0 + l
        s = g // BT
        bt = g - s * BT
        return (
            pltpu.make_async_copy(
                obuf.at[:, pl.ds(0, GSZ)],
                out3.at[s, :, pl.ds(bt * GSZ, GSZ)], sem),
            pltpu.make_async_copy(
                mbuf, mask_out.at[pl.ds(base + l * GSZ, GSZ)], sem),
        )

    iota16 = lax.iota(jnp.int32, LANES)
    hvecs = [iota16 + j * LANES for j in range(HIDDEN // LANES)]
    for p in range(NBUF):
        start_gather(p, gbufs[p], gsems[p])

    def quad_body(qq, carry):
        for q in range(NBUF):
            l = qq * NBUF + q
            gbuf, obuf, mbuf = gbufs[q], obufs[q % 2], mask_bs[q % 2]
            @pl.when(l >= 2)
            def _():
                for h in out_copy(l - 2, obuf, mbuf, osems[q % 2]):
                    h.wait()
            for i in range(GSZ // LANES):
                v = idx_all[pl.ds(l * GSZ + i * LANES, LANES)]
                ispad = v == PAD_IDX
                mbuf[pl.ds(i * LANES, LANES)] = jnp.where(ispad, 1, 0)
                np_b[pl.ds(i * LANES, LANES)] = jnp.where(ispad, 0.0, 1.0)
            drain_gather(l, gbuf, gsems[q])
            s = (g0 + l) // BT
            pvs = [pos_v[pl.ds(s * HIDDEN + j * LANES, LANES)]
                   for j in range(HIDDEN // LANES)]

            def row_body(r0, carry2):
                for u in range(RUNROLL):
                    r = r0 * RUNROLL + u
                    nps = np_b[pl.ds(r, LANES)][0]
                    rvec = jnp.broadcast_to(r, (LANES,))
                    for j in range(HIDDEN // LANES):
                        gv = gbuf[r, pl.ds(j * LANES, LANES)]
                        plsc.store_scatter(obuf, [hvecs[j], rvec],
                                           (gv + pvs[j]) * nps)
                return carry2

            lax.fori_loop(0, GSZ // RUNROLL, row_body, 0, unroll=False)
            for h in out_copy(l, obuf, mbuf, osems[q % 2]):
                h.start()
            @pl.when(l + NBUF < NG)
            def _():
                start_gather(l + NBUF, gbuf, gsems[q])
        return carry

    lax.fori_loop(0, NG // NBUF, quad_body, 0, unroll=False)
    for h in out_copy(NG - 2, obufs[0], mask_bs[0], osems[0]):
        h.wait()
    for h in out_copy(NG - 1, obufs[1], mask_bs[1], osems[1]):
        h.wait()


_emb_call = pl.kernel(
    _emb_body,
    out_type=[
        jax.ShapeDtypeStruct((SEQ, HIDDEN, BATCH), jnp.float32),
        jax.ShapeDtypeStruct((N,), jnp.int32),
    ],
    mesh=plsc.VectorSubcoreMesh(
        core_axis_name="c", subcore_axis_name="s", num_cores=NC,
        num_subcores=NS),
    scratch_types=[
        pltpu.VMEM((ROWS_PW,), jnp.int32),
        pltpu.VMEM((SEQ * HIDDEN + LANES,), jnp.float32),
        [pltpu.VMEM((GSZ, PADH), jnp.float32)] * NBUF,
        [pltpu.VMEM((HIDDEN, OST), jnp.float32)] * 2,
        [pltpu.VMEM((GSZ,), jnp.int32)] * 2,
        pltpu.VMEM((GSZ + LANES,), jnp.float32),
        [pltpu.SemaphoreType.DMA] * NBUF,
        [pltpu.SemaphoreType.DMA] * 2,
    ],
    compiler_params=pltpu.CompilerParams(needs_layout_passes=False),
)


def kernel(x, item_table, pos_table):
    xf = x.T.reshape(N)
    tp = jnp.pad(item_table, ((0, 0), (0, PADH - HIDDEN)))
    posf = jnp.pad(pos_table.reshape(SEQ * HIDDEN), (0, LANES))
    out3, mask_i32 = _emb_call(xf, tp, posf)
    input_emb = out3.transpose(2, 0, 1)
    pad_masking = mask_i32.reshape(SEQ, BATCH).T.astype(bool)
    return (input_emb, pad_masking)
